# baseline (speedup 1.0000x reference)
import functools

import jax
import jax.numpy as jnp
import numpy as np
from jax import lax
from jax.experimental import pallas as pl
from jax.experimental.pallas import tpu as pltpu
from jax.experimental.pallas import tpu_sc as plsc

DIM = 1024
HEADS = 8
NUM_KEYS = 256
TOPK = 32
B = 2
T = 2048
D_HEAD = DIM // HEADS
BT = B * T
HP = HEADS * 2
NEG = -1e30

_PAIRS = [(k1, k2) for k1 in range(TOPK) for k2 in range(TOPK // (k1 + 1))]
_NCAND = 128
_P1 = np.zeros((TOPK, _NCAND), np.float32)
_P2 = np.zeros((TOPK, _NCAND), np.float32)
_PADROW = np.full((1, _NCAND), NEG, np.float32)
for _j, (_k1, _k2) in enumerate(_PAIRS):
    _P1[_k1, _j] = 1.0
    _P2[_k2, _j] = 1.0
    _PADROW[0, _j] = 0.0

R1 = 512
I2 = 256

NW = 32
ROWS_PER_W = (B * 2 * HEADS * 1024) // NW
CHUNK = 128


def _topk_iter(vals, n, k, extra=None):
    iota = lax.broadcasted_iota(jnp.int32, (1, n), 1).astype(jnp.float32)
    ss, ii, ee = [], [], []
    for _ in range(k):
        m = jnp.max(vals, axis=-1, keepdims=True)
        sel = jnp.min(jnp.where(vals == m, iota, 1e9), axis=-1, keepdims=True)
        hit = iota == sel
        ss.append(m)
        ii.append(sel)
        if extra is not None:
            ee.append(jnp.sum(jnp.where(hit, extra, 0.0), axis=-1, keepdims=True))
        vals = jnp.where(hit, NEG, vals)
    s = jnp.concatenate(ss, axis=-1)
    i = jnp.concatenate(ii, axis=-1)
    if extra is not None:
        return s, i, jnp.concatenate(ee, axis=-1)
    return s, i


def _stage_a_kernel(x_ref, wq_ref, kt_ref, s_ref, i_ref):
    q = jnp.dot(x_ref[...].astype(jnp.bfloat16), wq_ref[0].astype(jnp.bfloat16),
                preferred_element_type=jnp.float32)
    dots = jnp.dot(q.astype(jnp.bfloat16), kt_ref[0].astype(jnp.bfloat16),
                   preferred_element_type=jnp.float32)
    s, i = _topk_iter(dots, NUM_KEYS, TOPK)
    s_ref[0] = s
    i_ref[0] = i


def _stage_a(x2d, wq_r, keys_t):
    return pl.pallas_call(
        _stage_a_kernel,
        grid=(HP, BT // R1),
        in_specs=[
            pl.BlockSpec((R1, DIM), lambda hp, rb: (rb, 0)),
            pl.BlockSpec((1, DIM, D_HEAD), lambda hp, rb: (hp, 0, 0)),
            pl.BlockSpec((1, D_HEAD, NUM_KEYS), lambda hp, rb: (hp, 0, 0)),
        ],
        out_specs=[
            pl.BlockSpec((1, R1, TOPK), lambda hp, rb: (hp, rb, 0)),
            pl.BlockSpec((1, R1, TOPK), lambda hp, rb: (hp, rb, 0)),
        ],
        out_shape=[
            jax.ShapeDtypeStruct((HP, BT, TOPK), jnp.float32),
            jax.ShapeDtypeStruct((HP, BT, TOPK), jnp.float32),
        ],
    )(x2d, wq_r, keys_t)


def _stage_b_kernel(s0l_ref, s0h_ref, s1l_ref, s1h_ref,
                    i0l_ref, i0h_ref, i1l_ref, i1h_ref,
                    p1_ref, p2_ref, pad_ref, w_ref, g_ref):
    h = pl.program_id(1)
    p1 = p1_ref[...]
    p2 = p2_ref[...]
    pad = pad_ref[...]
    for p, (sl, sh, il, ih) in enumerate((
            (s0l_ref, s0h_ref, i0l_ref, i0h_ref),
            (s1l_ref, s1h_ref, i1l_ref, i1h_ref))):
        cand = (jnp.dot(sl[0], p1, preferred_element_type=jnp.float32,
                        precision=lax.Precision.HIGHEST)
                + jnp.dot(sh[0], p2, preferred_element_type=jnp.float32,
                          precision=lax.Precision.HIGHEST) + pad)
        icand = (jnp.dot(il[0], p1, preferred_element_type=jnp.float32) * NUM_KEYS
                 + jnp.dot(ih[0], p2, preferred_element_type=jnp.float32))
        s, _, g = _topk_iter(cand, _NCAND, TOPK, extra=icand)
        e = jnp.exp(s - s[:, 0:1])
        attn = e / jnp.sum(e, axis=-1, keepdims=True)
        w_ref[0, p, 0] = attn
        g_ref[0, p, 0] = g.astype(jnp.int32) + h * (NUM_KEYS * NUM_KEYS)


def _stage_b(s_arr, i_arr, p1t, p2t, padrow):
    nb_lo = T // I2
    nb_hi = 1024 // I2

    def sspec(p, hi):
        return pl.BlockSpec(
            (1, I2, TOPK),
            lambda b, h, ib, p=p, hi=hi: (p * HEADS + h, b * nb_lo + hi * nb_hi + ib, 0))

    specs = [sspec(0, 0), sspec(0, 1), sspec(1, 0), sspec(1, 1)]
    return pl.pallas_call(
        _stage_b_kernel,
        grid=(B, HEADS, 1024 // I2),
        in_specs=specs + specs + [
            pl.BlockSpec((TOPK, _NCAND), lambda b, h, ib: (0, 0)),
            pl.BlockSpec((TOPK, _NCAND), lambda b, h, ib: (0, 0)),
            pl.BlockSpec((1, _NCAND), lambda b, h, ib: (0, 0)),
        ],
        out_specs=[
            pl.BlockSpec((1, 2, 1, I2, TOPK), lambda b, h, ib: (b, 0, h, ib, 0)),
            pl.BlockSpec((1, 2, 1, I2, TOPK), lambda b, h, ib: (b, 0, h, ib, 0)),
        ],
        out_shape=[
            jax.ShapeDtypeStruct((B, 2, HEADS, 1024, TOPK), jnp.float32),
            jax.ShapeDtypeStruct((B, 2, HEADS, 1024, TOPK), jnp.int32),
        ],
    )(s_arr, s_arr, s_arr, s_arr, i_arr, i_arr, i_arr, i_arr, p1t, p2t, padrow)


def _sc_gather_kernel(gidx_hbm, w_hbm, values_hbm, out_hbm,
                      idx_all, w_all, gbuf0, gbuf1, gbuf2, gbuf3, out_buf,
                      lsem, sem0, sem1, sem2, sem3):
    wid = lax.axis_index("s") * 2 + lax.axis_index("c")
    b = wid // 16
    wloc = wid % 16

    descs = []
    for p in range(2):
        for h in range(HEADS):
            ph = p * HEADS + h
            descs.append(pltpu.async_copy(
                gidx_hbm.at[b, p, h, pl.ds(wloc * 16, 16)],
                idx_all.at[pl.ds(ph * 16, 16)], lsem))
            descs.append(pltpu.async_copy(
                w_hbm.at[b, p, h, pl.ds(wloc * 16, 16)],
                w_all.at[pl.ds(ph * 16, 16)], lsem))
    for d in descs:
        d.wait()

    NG = 512

    def gmap(g):
        iblk = g // 64
        sub = g % 64
        ph = sub % 16
        qh = sub // 16
        rg = ph * 16 + iblk * 2 + qh // 2
        off = (qh % 2) * 64
        return rg, off, ph, qh * 2

    def issue(g, gbuf, sem):
        rg, off, _, _ = gmap(jnp.minimum(g, NG - 1))
        return pltpu.async_copy(
            values_hbm.at[idx_all.at[rg, pl.ds(off, 64)]], gbuf, sem)

    def wait(gbuf, sem):
        pltpu.make_async_copy(values_hbm.at[pl.ds(0, 64)], gbuf, sem).wait()

    def compute(g, gbuf):
        rg, off, ph, io8 = gmap(g)
        for r in range(2):
            w0 = w_all[rg, pl.ds(off + r * 32, 16)]
            w1 = w_all[rg, pl.ds(off + r * 32 + 16, 16)]
            ws = [w0[j] for j in range(16)] + [w1[j] for j in range(16)]
            out_row = (io8 + r) * 16 + ph
            for c in range(D_HEAD // 16):
                acc = ws[0] * gbuf[r * 32, pl.ds(c * 16, 16)]
                for j in range(1, TOPK):
                    acc = acc + ws[j] * gbuf[r * 32 + j, pl.ds(c * 16, 16)]
                out_buf[out_row, pl.ds(c * 16, 16)] = acc

    def maybe_flush(g):
        @pl.when(g % 64 == 63)
        def _():
            base = pl.multiple_of(
                b * 16384 + wloc * 1024 + (g // 64) * CHUNK, CHUNK)
            pltpu.sync_copy(out_buf, out_hbm.at[pl.ds(base, CHUNK)])

    bufs = (gbuf0, gbuf1, gbuf2, gbuf3)
    sems = (sem0, sem1, sem2, sem3)
    for l in range(3):
        issue(l, bufs[l], sems[l])

    def body(it, carry):
        g = it * 4
        for l in range(4):
            wait(bufs[l], sems[l])
            issue(g + l + 3, bufs[(l + 3) % 4], sems[(l + 3) % 4])
            compute(g + l, bufs[l])
            maybe_flush(g + l)
        return carry

    lax.fori_loop(0, NG // 4, body, 0)
    for l in range(3):
        wait(bufs[l], sems[l])


def _sc_gather(gidx, w, values_flat):
    mesh = plsc.VectorSubcoreMesh(core_axis_name="c", subcore_axis_name="s")
    kfn = functools.partial(
        pl.kernel,
        out_type=jax.ShapeDtypeStruct((B * T * HEADS, D_HEAD), jnp.float32),
        mesh=mesh,
        scratch_types=[
            pltpu.VMEM((256, 128), jnp.int32),
            pltpu.VMEM((256, 128), jnp.float32),
            pltpu.VMEM((64, D_HEAD), jnp.float32),
            pltpu.VMEM((64, D_HEAD), jnp.float32),
            pltpu.VMEM((64, D_HEAD), jnp.float32),
            pltpu.VMEM((64, D_HEAD), jnp.float32),
            pltpu.VMEM((CHUNK, D_HEAD), jnp.float32),
            pltpu.SemaphoreType.DMA,
            pltpu.SemaphoreType.DMA,
            pltpu.SemaphoreType.DMA,
            pltpu.SemaphoreType.DMA,
            pltpu.SemaphoreType.DMA,
        ],
    )(_sc_gather_kernel)
    return kfn(gidx, w, values_flat)


def _out_proj_kernel(x_ref, w_ref, b_ref, o_ref):
    o_ref[...] = jnp.dot(x_ref[...], w_ref[...],
                         preferred_element_type=jnp.float32) + b_ref[...]


def _out_proj(x2d, W_o, b_o):
    n, d = x2d.shape
    blk = 1024
    return pl.pallas_call(
        _out_proj_kernel,
        grid=(n // blk,),
        in_specs=[
            pl.BlockSpec((blk, d), lambda i: (i, 0)),
            pl.BlockSpec((d, d), lambda i: (0, 0)),
            pl.BlockSpec((1, d), lambda i: (0, 0)),
        ],
        out_specs=pl.BlockSpec((blk, d), lambda i: (i, 0)),
        out_shape=jax.ShapeDtypeStruct((n, d), jnp.float32),
    )(x2d, W_o.T, b_o[None, :])


def kernel(x, W_q, keys_p, values, W_o, b_o):
    b, t, e = x.shape
    x2d = x.reshape(BT, DIM)
    wq_r = W_q.reshape(2, HEADS, D_HEAD, DIM).transpose(0, 1, 3, 2).reshape(
        HP, DIM, D_HEAD)
    keys_t = keys_p.transpose(2, 0, 3, 1).reshape(HP, D_HEAD, NUM_KEYS)
    s_arr, i_arr = _stage_a(x2d, wq_r, keys_t)
    w_bt, g_bt = _stage_b(s_arr, i_arr, jnp.asarray(_P1), jnp.asarray(_P2),
                          jnp.asarray(_PADROW))
    values_flat = values.reshape(HEADS * NUM_KEYS * NUM_KEYS, D_HEAD)
    _DEBUG_JNP_GATHER = False
    if _DEBUG_JNP_GATHER:
        g2 = g_bt.reshape(-1, 32)
        w2 = w_bt.reshape(-1, 32)
        rows = values_flat[g2]
        o = (w2[..., None] * rows).sum(1).reshape(B, 2, HEADS, 1024, D_HEAD)
        out_sc = o.transpose(0, 3, 1, 2, 4).reshape(B * T * HEADS, D_HEAD)
    else:
        out_sc = _sc_gather(g_bt.reshape(B, 2, HEADS, 256, 128),
                            w_bt.reshape(B, 2, HEADS, 256, 128), values_flat)
    out = _out_proj(out_sc.reshape(BT, DIM), W_o, b_o)
    return out.reshape(b, t, e)


# speedup vs baseline: 1.2219x; 1.2219x over previous
import functools

import jax
import jax.numpy as jnp
import numpy as np
from jax import lax
from jax.experimental import pallas as pl
from jax.experimental.pallas import tpu as pltpu
from jax.experimental.pallas import tpu_sc as plsc

DIM = 1024
HEADS = 8
NUM_KEYS = 256
TOPK = 32
B = 2
T = 2048
D_HEAD = DIM // HEADS
BT = B * T
HP = HEADS * 2
NEG = -1e30

_PAIRS = [(k1, k2) for k1 in range(TOPK) for k2 in range(TOPK // (k1 + 1))]
_NCAND = 128
_P1 = np.zeros((TOPK, _NCAND), np.float32)
_P2 = np.zeros((TOPK, _NCAND), np.float32)
_PADROW = np.full((1, _NCAND), NEG, np.float32)
for _j, (_k1, _k2) in enumerate(_PAIRS):
    _P1[_k1, _j] = 1.0
    _P2[_k2, _j] = 1.0
    _PADROW[0, _j] = 0.0

R1 = 512
I2 = 256

NW = 32
ROWS_PER_W = (B * 2 * HEADS * 1024) // NW
CHUNK = 128


def _topk_iter(vals, n, k, extra=None):
    iota = lax.broadcasted_iota(jnp.int32, (1, n), 1).astype(jnp.float32)
    ss, ii, ee = [], [], []
    for _ in range(k):
        m = jnp.max(vals, axis=-1, keepdims=True)
        sel = jnp.min(jnp.where(vals == m, iota, 1e9), axis=-1, keepdims=True)
        hit = iota == sel
        ss.append(m)
        ii.append(sel)
        if extra is not None:
            ee.append(jnp.sum(jnp.where(hit, extra, 0.0), axis=-1, keepdims=True))
        vals = jnp.where(hit, NEG, vals)
    s = jnp.concatenate(ss, axis=-1)
    i = jnp.concatenate(ii, axis=-1)
    if extra is not None:
        return s, i, jnp.concatenate(ee, axis=-1)
    return s, i


def _stage_a_kernel(x_ref, wq_ref, kt_ref, s_ref, i_ref):
    q = jnp.dot(x_ref[...].astype(jnp.bfloat16), wq_ref[0].astype(jnp.bfloat16),
                preferred_element_type=jnp.float32)
    dots = jnp.dot(q.astype(jnp.bfloat16), kt_ref[0].astype(jnp.bfloat16),
                   preferred_element_type=jnp.float32)
    s, i = _topk_iter(dots, NUM_KEYS, TOPK)
    s_ref[0] = s
    i_ref[0] = i


def _stage_a(x2d, wq_r, keys_t):
    nrows = x2d.shape[0]
    return pl.pallas_call(
        _stage_a_kernel,
        grid=(HP, nrows // R1),
        in_specs=[
            pl.BlockSpec((R1, DIM), lambda hp, rb: (rb, 0)),
            pl.BlockSpec((1, DIM, D_HEAD), lambda hp, rb: (hp, 0, 0)),
            pl.BlockSpec((1, D_HEAD, NUM_KEYS), lambda hp, rb: (hp, 0, 0)),
        ],
        out_specs=[
            pl.BlockSpec((1, R1, TOPK), lambda hp, rb: (hp, rb, 0)),
            pl.BlockSpec((1, R1, TOPK), lambda hp, rb: (hp, rb, 0)),
        ],
        out_shape=[
            jax.ShapeDtypeStruct((HP, nrows, TOPK), jnp.float32),
            jax.ShapeDtypeStruct((HP, nrows, TOPK), jnp.float32),
        ],
    )(x2d, wq_r, keys_t)


def _stage_b_kernel(s0l_ref, s0h_ref, s1l_ref, s1h_ref,
                    i0l_ref, i0h_ref, i1l_ref, i1h_ref,
                    p1_ref, p2_ref, pad_ref, w_ref, g_ref):
    h = pl.program_id(1)
    p1 = p1_ref[...]
    p2 = p2_ref[...]
    pad = pad_ref[...]
    for p, (sl, sh, il, ih) in enumerate((
            (s0l_ref, s0h_ref, i0l_ref, i0h_ref),
            (s1l_ref, s1h_ref, i1l_ref, i1h_ref))):
        cand = (jnp.dot(sl[0], p1, preferred_element_type=jnp.float32,
                        precision=lax.Precision.HIGHEST)
                + jnp.dot(sh[0], p2, preferred_element_type=jnp.float32,
                          precision=lax.Precision.HIGHEST) + pad)
        icand = (jnp.dot(il[0], p1, preferred_element_type=jnp.float32) * NUM_KEYS
                 + jnp.dot(ih[0], p2, preferred_element_type=jnp.float32))
        s, _, g = _topk_iter(cand, _NCAND, TOPK, extra=icand)
        e = jnp.exp(s - s[:, 0:1])
        attn = e / jnp.sum(e, axis=-1, keepdims=True)
        w_ref[0, p, 0] = attn
        g_ref[0, p, 0] = g.astype(jnp.int32) + h * (NUM_KEYS * NUM_KEYS)


def _stage_b(s_arr, i_arr, p1t, p2t, padrow):
    nb = s_arr.shape[1] // T
    nb_lo = T // I2
    nb_hi = 1024 // I2

    def sspec(p, hi):
        return pl.BlockSpec(
            (1, I2, TOPK),
            lambda b, h, ib, p=p, hi=hi: (p * HEADS + h, b * nb_lo + hi * nb_hi + ib, 0))

    specs = [sspec(0, 0), sspec(0, 1), sspec(1, 0), sspec(1, 1)]
    return pl.pallas_call(
        _stage_b_kernel,
        grid=(nb, HEADS, 1024 // I2),
        in_specs=specs + specs + [
            pl.BlockSpec((TOPK, _NCAND), lambda b, h, ib: (0, 0)),
            pl.BlockSpec((TOPK, _NCAND), lambda b, h, ib: (0, 0)),
            pl.BlockSpec((1, _NCAND), lambda b, h, ib: (0, 0)),
        ],
        out_specs=[
            pl.BlockSpec((1, 2, 1, I2, TOPK), lambda b, h, ib: (b, 0, h, ib, 0)),
            pl.BlockSpec((1, 2, 1, I2, TOPK), lambda b, h, ib: (b, 0, h, ib, 0)),
        ],
        out_shape=[
            jax.ShapeDtypeStruct((nb, 2, HEADS, 1024, TOPK), jnp.float32),
            jax.ShapeDtypeStruct((nb, 2, HEADS, 1024, TOPK), jnp.int32),
        ],
    )(s_arr, s_arr, s_arr, s_arr, i_arr, i_arr, i_arr, i_arr, p1t, p2t, padrow)


def _sc_gather_kernel(gidx_hbm, w_hbm, values_hbm, out_hbm,
                      idx_all, w_all, gbuf0, gbuf1, gbuf2, gbuf3, out_buf,
                      lsem, sem0, sem1, sem2, sem3):
    wloc = lax.axis_index("s") * 2 + lax.axis_index("c")

    descs = []
    for p in range(2):
        for h in range(HEADS):
            ph = p * HEADS + h
            descs.append(pltpu.async_copy(
                gidx_hbm.at[p, h, pl.ds(wloc * 8, 8)],
                idx_all.at[pl.ds(ph * 8, 8)], lsem))
            descs.append(pltpu.async_copy(
                w_hbm.at[p, h, pl.ds(wloc * 8, 8)],
                w_all.at[pl.ds(ph * 8, 8)], lsem))
    for d in descs:
        d.wait()

    NG = 256

    def gmap(g):
        iblk = g // 64
        sub = g % 64
        ph = sub % 16
        qh = sub // 16
        rg = ph * 8 + iblk * 2 + qh // 2
        off = (qh % 2) * 64
        return rg, off, ph, qh * 2

    def issue(g, gbuf, sem):
        rg, off, _, _ = gmap(jnp.minimum(g, NG - 1))
        return pltpu.async_copy(
            values_hbm.at[idx_all.at[rg, pl.ds(off, 64)]], gbuf, sem)

    def wait(gbuf, sem):
        pltpu.make_async_copy(values_hbm.at[pl.ds(0, 64)], gbuf, sem).wait()

    def compute(g, gbuf):
        rg, off, ph, io8 = gmap(g)
        for r in range(2):
            w0 = w_all[rg, pl.ds(off + r * 32, 16)]
            w1 = w_all[rg, pl.ds(off + r * 32 + 16, 16)]
            ws = [w0[j] for j in range(16)] + [w1[j] for j in range(16)]
            out_row = (io8 + r) * 16 + ph
            for c in range(D_HEAD // 16):
                acc = ws[0] * gbuf[r * 32, pl.ds(c * 16, 16)]
                for j in range(1, TOPK):
                    acc = acc + ws[j] * gbuf[r * 32 + j, pl.ds(c * 16, 16)]
                out_buf[out_row, pl.ds(c * 16, 16)] = acc

    def maybe_flush(g):
        @pl.when(g % 64 == 63)
        def _():
            base = pl.multiple_of(wloc * 512 + (g // 64) * CHUNK, CHUNK)
            pltpu.sync_copy(out_buf, out_hbm.at[pl.ds(base, CHUNK)])

    bufs = (gbuf0, gbuf1, gbuf2, gbuf3)
    sems = (sem0, sem1, sem2, sem3)
    for l in range(3):
        issue(l, bufs[l], sems[l])

    def body(it, carry):
        g = it * 4
        for l in range(4):
            wait(bufs[l], sems[l])
            issue(g + l + 3, bufs[(l + 3) % 4], sems[(l + 3) % 4])
            compute(g + l, bufs[l])
            maybe_flush(g + l)
        return carry

    lax.fori_loop(0, NG // 4, body, 0)
    for l in range(3):
        wait(bufs[l], sems[l])


def _sc_gather(gidx, w, values_flat):
    mesh = plsc.VectorSubcoreMesh(core_axis_name="c", subcore_axis_name="s")
    kfn = functools.partial(
        pl.kernel,
        out_type=jax.ShapeDtypeStruct((T * HEADS, D_HEAD), jnp.float32),
        mesh=mesh,
        scratch_types=[
            pltpu.VMEM((128, 128), jnp.int32),
            pltpu.VMEM((128, 128), jnp.float32),
            pltpu.VMEM((64, D_HEAD), jnp.float32),
            pltpu.VMEM((64, D_HEAD), jnp.float32),
            pltpu.VMEM((64, D_HEAD), jnp.float32),
            pltpu.VMEM((64, D_HEAD), jnp.float32),
            pltpu.VMEM((CHUNK, D_HEAD), jnp.float32),
            pltpu.SemaphoreType.DMA,
            pltpu.SemaphoreType.DMA,
            pltpu.SemaphoreType.DMA,
            pltpu.SemaphoreType.DMA,
            pltpu.SemaphoreType.DMA,
        ],
    )(_sc_gather_kernel)
    return kfn(gidx, w, values_flat)


def _out_proj_kernel(x_ref, w_ref, b_ref, o_ref):
    o_ref[...] = jnp.dot(x_ref[...], w_ref[...],
                         preferred_element_type=jnp.float32) + b_ref[...]


def _out_proj(x2d, W_o, b_o):
    n, d = x2d.shape
    blk = 1024
    return pl.pallas_call(
        _out_proj_kernel,
        grid=(n // blk,),
        in_specs=[
            pl.BlockSpec((blk, d), lambda i: (i, 0)),
            pl.BlockSpec((d, d), lambda i: (0, 0)),
            pl.BlockSpec((1, d), lambda i: (0, 0)),
        ],
        out_specs=pl.BlockSpec((blk, d), lambda i: (i, 0)),
        out_shape=jax.ShapeDtypeStruct((n, d), jnp.float32),
    )(x2d, W_o.T, b_o[None, :])


def kernel(x, W_q, keys_p, values, W_o, b_o):
    b, t, e = x.shape
    x2d = x.reshape(BT, DIM)
    wq_r = W_q.reshape(2, HEADS, D_HEAD, DIM).transpose(0, 1, 3, 2).reshape(
        HP, DIM, D_HEAD)
    keys_t = keys_p.transpose(2, 0, 3, 1).reshape(HP, D_HEAD, NUM_KEYS)
    values_flat = values.reshape(HEADS * NUM_KEYS * NUM_KEYS, D_HEAD)
    p1c, p2c, padc = jnp.asarray(_P1), jnp.asarray(_P2), jnp.asarray(_PADROW)
    outs = []
    for bb in range(B):
        s_arr, i_arr = _stage_a(x2d[bb * T:(bb + 1) * T], wq_r, keys_t)
        w_bt, g_bt = _stage_b(s_arr, i_arr, p1c, p2c, padc)
        out_sc = _sc_gather(g_bt.reshape(2, HEADS, 256, 128),
                            w_bt.reshape(2, HEADS, 256, 128), values_flat)
        outs.append(_out_proj(out_sc.reshape(T, DIM), W_o, b_o))
    return jnp.stack(outs).reshape(b, t, e)


# speedup vs baseline: 1.7011x; 1.3922x over previous
import functools

import jax
import jax.numpy as jnp
import numpy as np
from jax import lax
from jax.experimental import pallas as pl
from jax.experimental.pallas import tpu as pltpu
from jax.experimental.pallas import tpu_sc as plsc

DIM = 1024
HEADS = 8
NUM_KEYS = 256
TOPK = 32
B = 2
T = 2048
D_HEAD = DIM // HEADS
BT = B * T
HP = HEADS * 2
NEG = -1e30

_PAIRS = [(k1, k2) for k1 in range(TOPK) for k2 in range(TOPK // (k1 + 1))]
_NCAND = 128
_P1 = np.zeros((TOPK, _NCAND), np.float32)
_P2 = np.zeros((TOPK, _NCAND), np.float32)
_PADROW = np.full((1, _NCAND), NEG, np.float32)
for _j, (_k1, _k2) in enumerate(_PAIRS):
    _P1[_k1, _j] = 1.0
    _P2[_k2, _j] = 1.0
    _PADROW[0, _j] = 0.0

R1 = 512
I2 = 256

NW = 32
ROWS_PER_W = (B * 2 * HEADS * 1024) // NW
CHUNK = 128


def _topk_iter_t(vals, n, k, extra=None):
    iota = lax.broadcasted_iota(jnp.int32, (n, 1), 0).astype(jnp.float32)

    def _reduce0(x, op, init_slab):
        slab = x[0:8]
        for s in range(1, n // 8):
            slab = op(slab, x[s * 8:(s + 1) * 8])
        return (jnp.max if op is jnp.maximum else jnp.min)(
            slab, axis=0, keepdims=True)

    ss, ii, ee = [], [], []
    for _ in range(k):
        m = _reduce0(vals, jnp.maximum, None)
        hit = jnp.where(vals == m, iota, 1e9)
        sel = _reduce0(hit, jnp.minimum, None)
        ss.append(m)
        ii.append(sel)
        if extra is not None:
            ee.append(_reduce0(jnp.where(iota == sel, extra, 1e9),
                               jnp.minimum, None))
        vals = jnp.where(iota == sel, NEG, vals)
    s = jnp.concatenate(ss, axis=0)
    i = jnp.concatenate(ii, axis=0)
    if extra is not None:
        return s, i, jnp.concatenate(ee, axis=0)
    return s, i


def _stage_a_kernel(x_ref, wq_ref, kt_ref, s_ref, i_ref):
    q = jnp.dot(x_ref[...].astype(jnp.bfloat16), wq_ref[0].astype(jnp.bfloat16),
                preferred_element_type=jnp.float32)
    dots = jnp.dot(q.astype(jnp.bfloat16), kt_ref[0].astype(jnp.bfloat16),
                   preferred_element_type=jnp.float32)
    s_t, i_t = _topk_iter_t(dots.T, NUM_KEYS, TOPK)
    s_ref[0] = s_t.T
    i_ref[0] = i_t.T


def _stage_a(x2d, wq_r, keys_t):
    nrows = x2d.shape[0]
    return pl.pallas_call(
        _stage_a_kernel,
        grid=(HP, nrows // R1),
        in_specs=[
            pl.BlockSpec((R1, DIM), lambda hp, rb: (rb, 0)),
            pl.BlockSpec((1, DIM, D_HEAD), lambda hp, rb: (hp, 0, 0)),
            pl.BlockSpec((1, D_HEAD, NUM_KEYS), lambda hp, rb: (hp, 0, 0)),
        ],
        out_specs=[
            pl.BlockSpec((1, R1, TOPK), lambda hp, rb: (hp, rb, 0)),
            pl.BlockSpec((1, R1, TOPK), lambda hp, rb: (hp, rb, 0)),
        ],
        out_shape=[
            jax.ShapeDtypeStruct((HP, nrows, TOPK), jnp.float32),
            jax.ShapeDtypeStruct((HP, nrows, TOPK), jnp.float32),
        ],
    )(x2d, wq_r, keys_t)


def _stage_b_kernel(s0l_ref, s0h_ref, s1l_ref, s1h_ref,
                    i0l_ref, i0h_ref, i1l_ref, i1h_ref,
                    p1_ref, p2_ref, pad_ref, w_ref, g_ref):
    h = pl.program_id(1)
    p1 = p1_ref[...]
    p2 = p2_ref[...]
    pad = pad_ref[...]
    for p, (sl, sh, il, ih) in enumerate((
            (s0l_ref, s0h_ref, i0l_ref, i0h_ref),
            (s1l_ref, s1h_ref, i1l_ref, i1h_ref))):
        candt = (jnp.dot(p1, sl[0].T, preferred_element_type=jnp.float32,
                         precision=lax.Precision.HIGHEST)
                 + jnp.dot(p2, sh[0].T, preferred_element_type=jnp.float32,
                           precision=lax.Precision.HIGHEST) + pad)
        icandt = (jnp.dot(p1, il[0].T, preferred_element_type=jnp.float32)
                  * NUM_KEYS
                  + jnp.dot(p2, ih[0].T, preferred_element_type=jnp.float32))
        s_t, _, g_t = _topk_iter_t(candt, _NCAND, TOPK, extra=icandt)
        e = jnp.exp(s_t - s_t[0:1])
        attn_t = e / jnp.sum(e, axis=0, keepdims=True)
        w_ref[0, p, 0] = attn_t.T
        g_ref[0, p, 0] = g_t.T.astype(jnp.int32) + h * (NUM_KEYS * NUM_KEYS)


def _stage_b(s_arr, i_arr, p1t, p2t, padrow):
    nb = s_arr.shape[1] // T
    nb_lo = T // I2
    nb_hi = 1024 // I2

    def sspec(p, hi):
        return pl.BlockSpec(
            (1, I2, TOPK),
            lambda b, h, ib, p=p, hi=hi: (p * HEADS + h, b * nb_lo + hi * nb_hi + ib, 0))

    specs = [sspec(0, 0), sspec(0, 1), sspec(1, 0), sspec(1, 1)]
    return pl.pallas_call(
        _stage_b_kernel,
        grid=(nb, HEADS, 1024 // I2),
        in_specs=specs + specs + [
            pl.BlockSpec((_NCAND, TOPK), lambda b, h, ib: (0, 0)),
            pl.BlockSpec((_NCAND, TOPK), lambda b, h, ib: (0, 0)),
            pl.BlockSpec((_NCAND, 1), lambda b, h, ib: (0, 0)),
        ],
        out_specs=[
            pl.BlockSpec((1, 2, 1, I2, TOPK), lambda b, h, ib: (b, 0, h, ib, 0)),
            pl.BlockSpec((1, 2, 1, I2, TOPK), lambda b, h, ib: (b, 0, h, ib, 0)),
        ],
        out_shape=[
            jax.ShapeDtypeStruct((nb, 2, HEADS, 1024, TOPK), jnp.float32),
            jax.ShapeDtypeStruct((nb, 2, HEADS, 1024, TOPK), jnp.int32),
        ],
    )(s_arr, s_arr, s_arr, s_arr, i_arr, i_arr, i_arr, i_arr, p1t, p2t, padrow)


def _sc_gather_kernel(gidx_hbm, w_hbm, values_hbm, out_hbm,
                      idx_all, w_all, gbuf0, gbuf1, gbuf2, gbuf3, out_buf,
                      lsem, sem0, sem1, sem2, sem3):
    wloc = lax.axis_index("s") * 2 + lax.axis_index("c")

    descs = []
    for p in range(2):
        for h in range(HEADS):
            ph = p * HEADS + h
            descs.append(pltpu.async_copy(
                gidx_hbm.at[p, h, pl.ds(wloc * 8, 8)],
                idx_all.at[pl.ds(ph * 8, 8)], lsem))
            descs.append(pltpu.async_copy(
                w_hbm.at[p, h, pl.ds(wloc * 8, 8)],
                w_all.at[pl.ds(ph * 8, 8)], lsem))
    for d in descs:
        d.wait()

    NG = 256

    def gmap(g):
        iblk = g // 64
        sub = g % 64
        ph = sub % 16
        qh = sub // 16
        rg = ph * 8 + iblk * 2 + qh // 2
        off = (qh % 2) * 64
        return rg, off, ph, qh * 2

    def issue(g, gbuf, sem):
        rg, off, _, _ = gmap(jnp.minimum(g, NG - 1))
        return pltpu.async_copy(
            values_hbm.at[idx_all.at[rg, pl.ds(off, 64)]], gbuf, sem)

    def wait(gbuf, sem):
        pltpu.make_async_copy(values_hbm.at[pl.ds(0, 64)], gbuf, sem).wait()

    def compute(g, gbuf):
        rg, off, ph, io8 = gmap(g)
        for r in range(2):
            w0 = w_all[rg, pl.ds(off + r * 32, 16)]
            w1 = w_all[rg, pl.ds(off + r * 32 + 16, 16)]
            ws = [w0[j] for j in range(16)] + [w1[j] for j in range(16)]
            out_row = (io8 + r) * 16 + ph
            for c in range(D_HEAD // 16):
                acc = ws[0] * gbuf[r * 32, pl.ds(c * 16, 16)]
                for j in range(1, TOPK):
                    acc = acc + ws[j] * gbuf[r * 32 + j, pl.ds(c * 16, 16)]
                out_buf[out_row, pl.ds(c * 16, 16)] = acc

    def maybe_flush(g):
        @pl.when(g % 64 == 63)
        def _():
            base = pl.multiple_of(wloc * 512 + (g // 64) * CHUNK, CHUNK)
            pltpu.sync_copy(out_buf, out_hbm.at[pl.ds(base, CHUNK)])

    bufs = (gbuf0, gbuf1, gbuf2, gbuf3)
    sems = (sem0, sem1, sem2, sem3)
    for l in range(3):
        issue(l, bufs[l], sems[l])

    def body(it, carry):
        g = it * 4
        for l in range(4):
            wait(bufs[l], sems[l])
            issue(g + l + 3, bufs[(l + 3) % 4], sems[(l + 3) % 4])
            compute(g + l, bufs[l])
            maybe_flush(g + l)
        return carry

    lax.fori_loop(0, NG // 4, body, 0)
    for l in range(3):
        wait(bufs[l], sems[l])


def _sc_gather(gidx, w, values_flat):
    mesh = plsc.VectorSubcoreMesh(core_axis_name="c", subcore_axis_name="s")
    kfn = functools.partial(
        pl.kernel,
        out_type=jax.ShapeDtypeStruct((T * HEADS, D_HEAD), jnp.float32),
        mesh=mesh,
        scratch_types=[
            pltpu.VMEM((128, 128), jnp.int32),
            pltpu.VMEM((128, 128), jnp.float32),
            pltpu.VMEM((64, D_HEAD), jnp.float32),
            pltpu.VMEM((64, D_HEAD), jnp.float32),
            pltpu.VMEM((64, D_HEAD), jnp.float32),
            pltpu.VMEM((64, D_HEAD), jnp.float32),
            pltpu.VMEM((CHUNK, D_HEAD), jnp.float32),
            pltpu.SemaphoreType.DMA,
            pltpu.SemaphoreType.DMA,
            pltpu.SemaphoreType.DMA,
            pltpu.SemaphoreType.DMA,
            pltpu.SemaphoreType.DMA,
        ],
    )(_sc_gather_kernel)
    return kfn(gidx, w, values_flat)


def _out_proj_kernel(x_ref, w_ref, b_ref, o_ref):
    o_ref[...] = jnp.dot(x_ref[...], w_ref[...],
                         preferred_element_type=jnp.float32) + b_ref[...]


def _out_proj(x2d, W_o, b_o):
    n, d = x2d.shape
    blk = 1024
    return pl.pallas_call(
        _out_proj_kernel,
        grid=(n // blk,),
        in_specs=[
            pl.BlockSpec((blk, d), lambda i: (i, 0)),
            pl.BlockSpec((d, d), lambda i: (0, 0)),
            pl.BlockSpec((1, d), lambda i: (0, 0)),
        ],
        out_specs=pl.BlockSpec((blk, d), lambda i: (i, 0)),
        out_shape=jax.ShapeDtypeStruct((n, d), jnp.float32),
    )(x2d, W_o.T, b_o[None, :])


def kernel(x, W_q, keys_p, values, W_o, b_o):
    b, t, e = x.shape
    x2d = x.reshape(BT, DIM)
    wq_r = W_q.reshape(2, HEADS, D_HEAD, DIM).transpose(0, 1, 3, 2).reshape(
        HP, DIM, D_HEAD)
    keys_t = keys_p.transpose(2, 0, 3, 1).reshape(HP, D_HEAD, NUM_KEYS)
    values_flat = values.reshape(HEADS * NUM_KEYS * NUM_KEYS, D_HEAD)
    p1c, p2c = jnp.asarray(_P1.T.copy()), jnp.asarray(_P2.T.copy())
    padc = jnp.asarray(_PADROW.T.copy())
    outs = []
    for bb in range(B):
        s_arr, i_arr = _stage_a(x2d[bb * T:(bb + 1) * T], wq_r, keys_t)
        w_bt, g_bt = _stage_b(s_arr, i_arr, p1c, p2c, padc)
        out_sc = _sc_gather(g_bt.reshape(2, HEADS, 256, 128),
                            w_bt.reshape(2, HEADS, 256, 128), values_flat)
        outs.append(_out_proj(out_sc.reshape(T, DIM), W_o, b_o))
    return jnp.stack(outs).reshape(b, t, e)
